# Initial kernel scaffold; baseline (speedup 1.0000x reference)
import jax
import jax.numpy as jnp
from jax.experimental import pallas as pl


def kernel(inputs, hx_k, support, gw0, gb0, W0, b0, R0, attw0, attb0, gw1, gb1, W1, b1, R1, attw1, attb1):
    raise NotImplementedError("write your pallas kernel here")


# speedup vs baseline: 2.0462x; 2.0462x over previous
import jax
import jax.numpy as jnp
from jax.experimental import pallas as pl
from jax.experimental.pallas import tpu as pltpu

N = 1024
D = 64
K = 3
NL = 2
BS = 64
BT = 2

_F32 = jnp.float32
_BF16 = jnp.bfloat16


def _body(inp_ref, hx_ref, sb_ref,
          wx0_ref, wh2_0_ref, wh1_0_ref, gb0_ref, W0_ref, b0_ref, R0_ref, aw0_ref,
          wx1_ref, wh2_1_ref, wh1_1_ref, gb1_ref, W1_ref, b1_ref, R1_ref, aw1_ref,
          out_ref, hxn_ref):
    Sb = sb_ref[...]

    def layer(t_in, h0, h1, h2, wh2_ref, wh1_ref, gb_ref, W_ref, b_ref, R_ref, aw_ref):
        t = (t_in
             + jnp.dot(h2, wh2_ref[...], preferred_element_type=_F32)
             + jnp.dot(h1, wh1_ref[...], preferred_element_type=_F32))
        ta = t[:, 0:D]
        tb = t[:, D:2 * D]
        tc = t[:, 2 * D:3 * D]
        u = jnp.dot(Sb, tc.astype(_BF16), preferred_element_type=_F32)
        v = jnp.dot(Sb, (tb + u).astype(_BF16), preferred_element_type=_F32)
        diff = ta + v + gb_ref[...]
        conv = jnp.where(diff >= 0, diff, 0.01 * diff)
        ns0 = h0 + R_ref[0]
        ns1 = h1 + R_ref[1]
        ns2 = h2 + R_ref[2]
        aw = aw_ref[...]
        s0 = jnp.sum(ns0 * aw)
        s1 = jnp.sum(ns1 * aw)
        s2 = jnp.sum(ns2 * aw)
        m = jnp.maximum(jnp.maximum(s0, s1), s2)
        e0 = jnp.exp(s0 - m)
        e1 = jnp.exp(s1 - m)
        e2 = jnp.exp(s2 - m)
        inv = 1.0 / (e0 + e1 + e2)
        att = (e0 * inv) * ns0 + (e1 * inv) * ns1 + (e2 * inv) * ns2
        return jnp.dot(conv, W_ref[...], preferred_element_type=_F32) + b_ref[...] + att

    for e in range(BT):
        xin = inp_ref[e]
        t_in0 = xin[:, 0:1] * wx0_ref[0:1, :] + xin[:, 1:2] * wx0_ref[1:2, :]
        h00 = hx_ref[0, e, 0]
        h01 = hx_ref[0, e, 1]
        h02 = hx_ref[0, e, 2]
        out0 = layer(t_in0, h00, h01, h02, wh2_0_ref, wh1_0_ref,
                     gb0_ref, W0_ref, b0_ref, R0_ref, aw0_ref)
        t_in1 = jnp.dot(out0, wx1_ref[...], preferred_element_type=_F32)
        h10 = hx_ref[1, e, 0]
        h11 = hx_ref[1, e, 1]
        h12 = hx_ref[1, e, 2]
        out1 = layer(t_in1, h10, h11, h12, wh2_1_ref, wh1_1_ref,
                     gb1_ref, W1_ref, b1_ref, R1_ref, aw1_ref)
        hxn_ref[0, e, 0] = h01
        hxn_ref[0, e, 1] = h02
        hxn_ref[0, e, 2] = out0
        hxn_ref[1, e, 0] = h11
        hxn_ref[1, e, 1] = h12
        hxn_ref[1, e, 2] = out1
        out_ref[e] = out1


def _prep(gw, in_dim):
    g = gw.reshape(in_dim + 2 * D, K, D)
    wa = jnp.concatenate([g[:, 0] - g[:, 2], g[:, 1], 2.0 * g[:, 2]], axis=1)
    return wa[:in_dim], wa[in_dim:in_dim + D], wa[in_dim + D:]


def kernel(inputs, hx_k, support, gw0, gb0, W0, b0, R0, attw0, attb0,
           gw1, gb1, W1, b1, R1, attw1, attb1):
    del attb0, attb1
    inp3 = inputs.reshape(BS, N, 2)
    Sb = support.astype(_BF16)
    wx0, wh2_0, wh1_0 = _prep(gw0, 2)
    wx1, wh2_1, wh1_1 = _prep(gw1, D)
    aw0 = attw0.reshape(N, D)
    aw1 = attw1.reshape(N, D)
    gb0r = gb0.reshape(1, D)
    gb1r = gb1.reshape(1, D)

    grid = (BS // BT,)
    const = lambda shape: pl.BlockSpec(shape, lambda i: (0,) * len(shape))
    in_specs = [
        pl.BlockSpec((BT, N, 2), lambda i: (i, 0, 0)),
        pl.BlockSpec((NL, BT, K, N, D), lambda i: (0, i, 0, 0, 0)),
        const((N, N)),
        const((2, K * D)), const((D, K * D)), const((D, K * D)),
        const((1, D)), const((D, D)), const((N, D)), const((K, N, D)), const((N, D)),
        const((D, K * D)), const((D, K * D)), const((D, K * D)),
        const((1, D)), const((D, D)), const((N, D)), const((K, N, D)), const((N, D)),
    ]
    out_specs = [
        pl.BlockSpec((BT, N, D), lambda i: (i, 0, 0)),
        pl.BlockSpec((NL, BT, K, N, D), lambda i: (0, i, 0, 0, 0)),
    ]
    out, hxn = pl.pallas_call(
        _body,
        grid=grid,
        in_specs=in_specs,
        out_specs=out_specs,
        out_shape=[
            jax.ShapeDtypeStruct((BS, N, D), _F32),
            jax.ShapeDtypeStruct((NL, BS, K, N, D), _F32),
        ],
        compiler_params=pltpu.CompilerParams(
            dimension_semantics=("parallel",),
        ),
    )(inp3, hx_k, Sb,
      wx0, wh2_0, wh1_0, gb0r, W0, b0, R0, aw0,
      wx1, wh2_1, wh1_1, gb1r, W1, b1, R1, aw1)
    return out.reshape(BS, N * D), hxn


# speedup vs baseline: 2.3000x; 1.1240x over previous
import jax
import jax.numpy as jnp
from jax.experimental import pallas as pl
from jax.experimental.pallas import tpu as pltpu

N = 1024
D = 64
K = 3
NL = 2
BS = 64
BT = 2

_F32 = jnp.float32
_BF16 = jnp.bfloat16


def _body(inp_ref, hx_ref, sb_ref,
          wx0_ref, wh2_0_ref, wh1_0_ref, gb0_ref, W0_ref, b0_ref, R0_ref, aw0_ref,
          wx1_ref, wh2_1_ref, wh1_1_ref, gb1_ref, W1_ref, b1_ref, R1_ref, aw1_ref,
          out_ref, hxn_ref):
    Sb = sb_ref[...]

    def layer(Tin, l, wh2_ref, wh1_ref, gb_ref, W_ref, b_ref, R_ref, aw_ref):
        H2 = jnp.concatenate([hx_ref[l, e, 2] for e in range(BT)], axis=0)
        H1 = jnp.concatenate([hx_ref[l, e, 1] for e in range(BT)], axis=0)
        T = (Tin
             + jnp.dot(H2, wh2_ref[...], preferred_element_type=_F32)
             + jnp.dot(H1, wh1_ref[...], preferred_element_type=_F32))
        TCc = jnp.concatenate(
            [T[e * N:(e + 1) * N, 2 * D:3 * D] for e in range(BT)], axis=1)
        U = jnp.dot(Sb, TCc.astype(_BF16), preferred_element_type=_F32)
        TBc = jnp.concatenate(
            [T[e * N:(e + 1) * N, D:2 * D] for e in range(BT)], axis=1) + U
        V = jnp.dot(Sb, TBc.astype(_BF16), preferred_element_type=_F32)
        convs = []
        for e in range(BT):
            diff = T[e * N:(e + 1) * N, 0:D] + V[:, e * D:(e + 1) * D] + gb_ref[...]
            convs.append(jnp.where(diff >= 0, diff, 0.01 * diff))
        CONV = jnp.dot(jnp.concatenate(convs, axis=0), W_ref[...],
                       preferred_element_type=_F32)
        outs = []
        aw = aw_ref[...]
        for e in range(BT):
            ns0 = hx_ref[l, e, 0] + R_ref[0]
            ns1 = hx_ref[l, e, 1] + R_ref[1]
            ns2 = hx_ref[l, e, 2] + R_ref[2]
            s0 = jnp.sum(ns0 * aw)
            s1 = jnp.sum(ns1 * aw)
            s2 = jnp.sum(ns2 * aw)
            m = jnp.maximum(jnp.maximum(s0, s1), s2)
            e0 = jnp.exp(s0 - m)
            e1 = jnp.exp(s1 - m)
            e2 = jnp.exp(s2 - m)
            inv = 1.0 / (e0 + e1 + e2)
            att = (e0 * inv) * ns0 + (e1 * inv) * ns1 + (e2 * inv) * ns2
            outs.append(CONV[e * N:(e + 1) * N] + b_ref[...] + att)
        return outs

    t_in0 = []
    for e in range(BT):
        xin = inp_ref[e]
        t_in0.append(xin[:, 0:1] * wx0_ref[0:1, :] + xin[:, 1:2] * wx0_ref[1:2, :])
    out0s = layer(jnp.concatenate(t_in0, axis=0), 0, wh2_0_ref, wh1_0_ref,
                  gb0_ref, W0_ref, b0_ref, R0_ref, aw0_ref)
    Tin1 = jnp.dot(jnp.concatenate(out0s, axis=0), wx1_ref[...],
                   preferred_element_type=_F32)
    out1s = layer(Tin1, 1, wh2_1_ref, wh1_1_ref,
                  gb1_ref, W1_ref, b1_ref, R1_ref, aw1_ref)
    for e in range(BT):
        hxn_ref[0, e, 0] = hx_ref[0, e, 1]
        hxn_ref[0, e, 1] = hx_ref[0, e, 2]
        hxn_ref[0, e, 2] = out0s[e]
        hxn_ref[1, e, 0] = hx_ref[1, e, 1]
        hxn_ref[1, e, 1] = hx_ref[1, e, 2]
        hxn_ref[1, e, 2] = out1s[e]
        out_ref[e] = out1s[e]


def _prep(gw, in_dim):
    g = gw.reshape(in_dim + 2 * D, K, D)
    wa = jnp.concatenate([g[:, 0] - g[:, 2], g[:, 1], 2.0 * g[:, 2]], axis=1)
    return wa[:in_dim], wa[in_dim:in_dim + D], wa[in_dim + D:]


def kernel(inputs, hx_k, support, gw0, gb0, W0, b0, R0, attw0, attb0,
           gw1, gb1, W1, b1, R1, attw1, attb1):
    del attb0, attb1
    inp3 = inputs.reshape(BS, N, 2)
    Sb = support.astype(_BF16)
    wx0, wh2_0, wh1_0 = _prep(gw0, 2)
    wx1, wh2_1, wh1_1 = _prep(gw1, D)
    aw0 = attw0.reshape(N, D)
    aw1 = attw1.reshape(N, D)
    gb0r = gb0.reshape(1, D)
    gb1r = gb1.reshape(1, D)

    grid = (BS // BT,)
    const = lambda shape: pl.BlockSpec(shape, lambda i: (0,) * len(shape))
    in_specs = [
        pl.BlockSpec((BT, N, 2), lambda i: (i, 0, 0)),
        pl.BlockSpec((NL, BT, K, N, D), lambda i: (0, i, 0, 0, 0)),
        const((N, N)),
        const((2, K * D)), const((D, K * D)), const((D, K * D)),
        const((1, D)), const((D, D)), const((N, D)), const((K, N, D)), const((N, D)),
        const((D, K * D)), const((D, K * D)), const((D, K * D)),
        const((1, D)), const((D, D)), const((N, D)), const((K, N, D)), const((N, D)),
    ]
    out_specs = [
        pl.BlockSpec((BT, N, D), lambda i: (i, 0, 0)),
        pl.BlockSpec((NL, BT, K, N, D), lambda i: (0, i, 0, 0, 0)),
    ]
    out, hxn = pl.pallas_call(
        _body,
        grid=grid,
        in_specs=in_specs,
        out_specs=out_specs,
        out_shape=[
            jax.ShapeDtypeStruct((BS, N, D), _F32),
            jax.ShapeDtypeStruct((NL, BS, K, N, D), _F32),
        ],
        compiler_params=pltpu.CompilerParams(
            dimension_semantics=("parallel",),
        ),
    )(inp3, hx_k, Sb,
      wx0, wh2_0, wh1_0, gb0r, W0, b0, R0, aw0,
      wx1, wh2_1, wh1_1, gb1r, W1, b1, R1, aw1)
    return out.reshape(BS, N * D), hxn
